# baseline (speedup 1.0000x reference)
import functools

import jax
import jax.numpy as jnp
from jax import lax
from jax.experimental import pallas as pl
from jax.experimental.pallas import tpu as pltpu
from jax.experimental.pallas import tpu_sc as plsc

E = 8
KTOP = 2
D = 1024
FF = 1408
SFF = 2816
LANES = 128

NC_SC = 2
NS_SC = 16
NW = NC_SC * NS_SC

BBLK = 128
TBLK = 256
NCH = 2
CH = SFF // NCH


def _gate_body(x_ref, gw_ref, w_ref, i_ref):
    logits = lax.dot_general(
        x_ref[...], gw_ref[...], (((1,), (1,)), ((), ())),
        preferred_element_type=jnp.float32)
    lane = lax.broadcasted_iota(jnp.int32, logits.shape, 1)
    valid = lane < E
    lm = jnp.where(valid, logits, jnp.float32(-1e30))
    m = jnp.max(lm, axis=1, keepdims=True)
    p = jnp.exp(lm - m)
    p = jnp.where(valid, p, 0.0)
    probs = p / jnp.sum(p, axis=1, keepdims=True)
    m1 = jnp.max(probs, axis=1, keepdims=True)
    i1 = jnp.min(jnp.where(probs == m1, lane, LANES), axis=1, keepdims=True)
    probs2 = jnp.where(lane == i1, -1.0, probs)
    m2 = jnp.max(probs2, axis=1, keepdims=True)
    i2 = jnp.min(jnp.where(probs2 == m2, lane, LANES), axis=1, keepdims=True)
    w_ref[...] = jnp.where(lane == 0, m1, jnp.where(lane == 1, m2, 0.0))
    i_ref[...] = jnp.where(lane == 0, i1, jnp.where(lane == 1, i2, 0))


def _moe_body(be_ref, xs_ref, guw_ref, dw_ref, ys_ref):
    del be_ref
    h = lax.dot_general(
        xs_ref[...].astype(jnp.bfloat16),
        guw_ref[0].astype(jnp.bfloat16), (((1,), (1,)), ((), ())),
        preferred_element_type=jnp.float32)
    g = h[:, :FF]
    u = h[:, FF:]
    act = (g * lax.logistic(g)) * u
    ys_ref[...] = lax.dot_general(
        act.astype(jnp.bfloat16), dw_ref[0].astype(jnp.bfloat16),
        (((1,), (1,)), ((), ())),
        preferred_element_type=jnp.float32)


def _shared_body(x_ref, sg_ref, su_ref, sd_ref, y_ref):
    c = pl.program_id(1)
    x = x_ref[...].astype(jnp.bfloat16)
    hg = lax.dot_general(x, sg_ref[...].astype(jnp.bfloat16),
                         (((1,), (1,)), ((), ())),
                         preferred_element_type=jnp.float32)
    hu = lax.dot_general(x, su_ref[...].astype(jnp.bfloat16),
                         (((1,), (1,)), ((), ())),
                         preferred_element_type=jnp.float32)
    act = (hg * lax.logistic(hg)) * hu
    part = lax.dot_general(act.astype(jnp.bfloat16),
                           sd_ref[...].astype(jnp.bfloat16),
                           (((1,), (1,)), ((), ())),
                           preferred_element_type=jnp.float32)

    @pl.when(c == 0)
    def _init():
        y_ref[...] = part

    @pl.when(c != 0)
    def _acc():
        y_ref[...] += part


def _combine_body(ysh_ref, z_ref, w_ref, y_ref):
    w = w_ref[...]
    lane = lax.broadcasted_iota(jnp.int32, w.shape, 1)
    w0 = jnp.sum(jnp.where(lane == 0, w, 0.0), axis=1, keepdims=True)
    w1 = jnp.sum(jnp.where(lane == 1, w, 0.0), axis=1, keepdims=True)
    z = z_ref[...]
    y_ref[...] = ysh_ref[...] + w0 * z[:, :D] + w1 * z[:, D:]


def _make_sc_gather(rows_out, chunk):
    per_w = rows_out // NW
    n_chunks = per_w // chunk
    mesh = plsc.VectorSubcoreMesh(
        core_axis_name="c", subcore_axis_name="s",
        num_cores=NC_SC, num_subcores=NS_SC)

    @functools.partial(
        pl.kernel,
        out_type=jax.ShapeDtypeStruct((rows_out, D), jnp.float32),
        mesh=mesh,
        scratch_types=[
            pltpu.VMEM((chunk,), jnp.int32),
            pltpu.VMEM((chunk,), jnp.int32),
            pltpu.VMEM((chunk, D), jnp.float32),
            pltpu.VMEM((chunk, D), jnp.float32),
            pltpu.SemaphoreType.DMA,
            pltpu.SemaphoreType.DMA,
        ],
    )
    def gather_k(table_hbm, idx_hbm, out_hbm, idx_v0, idx_v1, rows_v0,
                 rows_v1, sem_g, sem_w):
        wid = lax.axis_index("s") * NC_SC + lax.axis_index("c")
        idx_bufs = (idx_v0, idx_v1)
        row_bufs = (rows_v0, rows_v1)
        base0 = wid * per_w
        pltpu.sync_copy(idx_hbm.at[pl.ds(base0, chunk)], idx_bufs[0])
        g_handle = pltpu.async_copy(
            table_hbm.at[idx_bufs[0]], row_bufs[0], sem_g)
        w_handles = [None, None]
        for c in range(n_chunks):
            cur = c % 2
            nxt = 1 - cur
            if c + 1 < n_chunks:
                pltpu.sync_copy(
                    idx_hbm.at[pl.ds(base0 + (c + 1) * chunk, chunk)],
                    idx_bufs[nxt])
            g_handle.wait()
            if c + 1 < n_chunks:
                if w_handles[nxt] is not None:
                    w_handles[nxt].wait()
                g_handle = pltpu.async_copy(
                    table_hbm.at[idx_bufs[nxt]], row_bufs[nxt], sem_g)
            w_handles[cur] = pltpu.async_copy(
                row_bufs[cur], out_hbm.at[pl.ds(base0 + c * chunk, chunk)],
                sem_w)
        for h in w_handles:
            if h is not None:
                h.wait()

    return gather_k


def kernel(hidden_states, gate_w, gate_up_w, down_w, shared_gate_up_w,
           shared_down_w):
    orig_shape = hidden_states.shape
    x = hidden_states.reshape(-1, D)
    t_tokens = x.shape[0]
    nslot = t_tokens * KTOP
    g_blocks = nslot // BBLK + E
    rows = g_blocks * BBLK

    gw_pad = jnp.zeros((LANES, D), jnp.float32).at[:E].set(gate_w)
    wpad, ipad = pl.pallas_call(
        _gate_body,
        grid=(t_tokens // TBLK,),
        in_specs=[
            pl.BlockSpec((TBLK, D), lambda t: (t, 0)),
            pl.BlockSpec((LANES, D), lambda t: (0, 0)),
        ],
        out_specs=[
            pl.BlockSpec((TBLK, LANES), lambda t: (t, 0)),
            pl.BlockSpec((TBLK, LANES), lambda t: (t, 0)),
        ],
        out_shape=[
            jax.ShapeDtypeStruct((t_tokens, LANES), jnp.float32),
            jax.ShapeDtypeStruct((t_tokens, LANES), jnp.int32),
        ],
    )(x, gw_pad)

    e_flat = ipad[:, :KTOP].reshape(-1)
    one_hot = (e_flat[:, None] == jnp.arange(E)[None, :]).astype(jnp.int32)
    csum = jnp.cumsum(one_hot, axis=0)
    rank_within = jnp.take_along_axis(
        csum, e_flat[:, None], axis=1)[:, 0] - 1
    sizes = csum[-1]
    nb = (sizes + BBLK - 1) // BBLK
    cum_nb = jnp.cumsum(nb)
    block_first = cum_nb - nb
    pos_of_slot = (block_first[e_flat] * BBLK + rank_within
                   ).astype(jnp.int32)
    tok_pad = (jnp.arange(rows, dtype=jnp.int32) % t_tokens).at[
        pos_of_slot].set(
            (jnp.arange(nslot, dtype=jnp.int32) // KTOP))
    block_expert = jnp.clip(
        jnp.searchsorted(cum_nb, jnp.arange(g_blocks), side="right"),
        0, E - 1).astype(jnp.int32)

    xs = _make_sc_gather(rows, 40)(x, tok_pad)

    grid_spec = pltpu.PrefetchScalarGridSpec(
        num_scalar_prefetch=1,
        grid=(g_blocks,),
        in_specs=[
            pl.BlockSpec((BBLK, D), lambda g, be: (g, 0)),
            pl.BlockSpec((1, 2 * FF, D), lambda g, be: (be[g], 0, 0)),
            pl.BlockSpec((1, D, FF), lambda g, be: (be[g], 0, 0)),
        ],
        out_specs=pl.BlockSpec((BBLK, D), lambda g, be: (g, 0)),
    )
    ys = pl.pallas_call(
        _moe_body,
        grid_spec=grid_spec,
        out_shape=jax.ShapeDtypeStruct((rows, D), jnp.float32),
        compiler_params=pltpu.CompilerParams(
            dimension_semantics=("arbitrary",)),
    )(block_expert, xs, gate_up_w, down_w)

    z = _make_sc_gather(nslot, 32)(ys, pos_of_slot)
    z2 = z.reshape(t_tokens, KTOP * D)

    sgw = shared_gate_up_w[:SFF]
    suw = shared_gate_up_w[SFF:]
    ysh = pl.pallas_call(
        _shared_body,
        grid=(t_tokens // TBLK, NCH),
        in_specs=[
            pl.BlockSpec((TBLK, D), lambda t, c: (t, 0)),
            pl.BlockSpec((CH, D), lambda t, c: (c, 0)),
            pl.BlockSpec((CH, D), lambda t, c: (c, 0)),
            pl.BlockSpec((D, CH), lambda t, c: (0, c)),
        ],
        out_specs=pl.BlockSpec((TBLK, D), lambda t, c: (t, 0)),
        out_shape=jax.ShapeDtypeStruct((t_tokens, D), jnp.float32),
        compiler_params=pltpu.CompilerParams(
            dimension_semantics=("parallel", "arbitrary")),
    )(x, sgw, suw, shared_down_w)

    y = pl.pallas_call(
        _combine_body,
        grid=(t_tokens // TBLK,),
        in_specs=[
            pl.BlockSpec((TBLK, D), lambda t: (t, 0)),
            pl.BlockSpec((TBLK, KTOP * D), lambda t: (t, 0)),
            pl.BlockSpec((TBLK, LANES), lambda t: (t, 0)),
        ],
        out_specs=pl.BlockSpec((TBLK, D), lambda t: (t, 0)),
        out_shape=jax.ShapeDtypeStruct((t_tokens, D), jnp.float32),
    )(ysh, z2, wpad)

    return y.reshape(orig_shape)


# speedup vs baseline: 1.0025x; 1.0025x over previous
import functools

import jax
import jax.numpy as jnp
from jax import lax
from jax.experimental import pallas as pl
from jax.experimental.pallas import tpu as pltpu
from jax.experimental.pallas import tpu_sc as plsc

E = 8
KTOP = 2
D = 1024
FF = 1408
SFF = 2816
LANES = 128

NC_SC = 2
NS_SC = 16
NW = NC_SC * NS_SC

BBLK = 128
TBLK = 256
NCH = 2
CH = SFF // NCH


def _gate_body(x_ref, gw_ref, w_ref, i_ref):
    logits = lax.dot_general(
        x_ref[...], gw_ref[...], (((1,), (1,)), ((), ())),
        preferred_element_type=jnp.float32)
    lane = lax.broadcasted_iota(jnp.int32, logits.shape, 1)
    valid = lane < E
    lm = jnp.where(valid, logits, jnp.float32(-1e30))
    m = jnp.max(lm, axis=1, keepdims=True)
    p = jnp.exp(lm - m)
    p = jnp.where(valid, p, 0.0)
    probs = p / jnp.sum(p, axis=1, keepdims=True)
    m1 = jnp.max(probs, axis=1, keepdims=True)
    i1 = jnp.min(jnp.where(probs == m1, lane, LANES), axis=1, keepdims=True)
    probs2 = jnp.where(lane == i1, -1.0, probs)
    m2 = jnp.max(probs2, axis=1, keepdims=True)
    i2 = jnp.min(jnp.where(probs2 == m2, lane, LANES), axis=1, keepdims=True)
    w_ref[...] = jnp.where(lane == 0, m1, jnp.where(lane == 1, m2, 0.0))
    i_ref[...] = jnp.where(lane == 0, i1, jnp.where(lane == 1, i2, 0))


def _moe_body(be_ref, xs_ref, guw_ref, dw_ref, ys_ref):
    del be_ref
    h = lax.dot_general(
        xs_ref[...], guw_ref[0], (((1,), (1,)), ((), ())),
        preferred_element_type=jnp.float32)
    g = h[:, :FF]
    u = h[:, FF:]
    act = (g * lax.logistic(g)) * u
    ys_ref[...] = lax.dot_general(
        act, dw_ref[0], (((1,), (1,)), ((), ())),
        preferred_element_type=jnp.float32)


def _shared_body(x_ref, sg_ref, su_ref, sd_ref, y_ref):
    c = pl.program_id(1)
    x = x_ref[...]
    hg = lax.dot_general(x, sg_ref[...], (((1,), (1,)), ((), ())),
                         preferred_element_type=jnp.float32)
    hu = lax.dot_general(x, su_ref[...], (((1,), (1,)), ((), ())),
                         preferred_element_type=jnp.float32)
    act = (hg * lax.logistic(hg)) * hu
    part = lax.dot_general(act, sd_ref[...], (((1,), (1,)), ((), ())),
                           preferred_element_type=jnp.float32)

    @pl.when(c == 0)
    def _init():
        y_ref[...] = part

    @pl.when(c != 0)
    def _acc():
        y_ref[...] += part


def _combine_body(ysh_ref, z_ref, w_ref, y_ref):
    w = w_ref[...]
    lane = lax.broadcasted_iota(jnp.int32, w.shape, 1)
    w0 = jnp.sum(jnp.where(lane == 0, w, 0.0), axis=1, keepdims=True)
    w1 = jnp.sum(jnp.where(lane == 1, w, 0.0), axis=1, keepdims=True)
    z = z_ref[...]
    y_ref[...] = ysh_ref[...] + w0 * z[:, :D] + w1 * z[:, D:]


def _make_sc_gather(rows_out, chunk):
    per_w = rows_out // NW
    n_chunks = per_w // chunk
    mesh = plsc.VectorSubcoreMesh(
        core_axis_name="c", subcore_axis_name="s",
        num_cores=NC_SC, num_subcores=NS_SC)

    @functools.partial(
        pl.kernel,
        out_type=jax.ShapeDtypeStruct((rows_out, D), jnp.float32),
        mesh=mesh,
        scratch_types=[
            pltpu.VMEM((chunk,), jnp.int32),
            pltpu.VMEM((chunk,), jnp.int32),
            pltpu.VMEM((chunk, D), jnp.float32),
            pltpu.VMEM((chunk, D), jnp.float32),
            pltpu.SemaphoreType.DMA,
            pltpu.SemaphoreType.DMA,
        ],
    )
    def gather_k(table_hbm, idx_hbm, out_hbm, idx_v0, idx_v1, rows_v0,
                 rows_v1, sem_g, sem_w):
        wid = lax.axis_index("s") * NC_SC + lax.axis_index("c")
        idx_bufs = (idx_v0, idx_v1)
        row_bufs = (rows_v0, rows_v1)
        base0 = wid * per_w
        pltpu.sync_copy(idx_hbm.at[pl.ds(base0, chunk)], idx_bufs[0])
        g_handle = pltpu.async_copy(
            table_hbm.at[idx_bufs[0]], row_bufs[0], sem_g)
        w_handles = [None, None]
        for c in range(n_chunks):
            cur = c % 2
            nxt = 1 - cur
            if c + 1 < n_chunks:
                pltpu.sync_copy(
                    idx_hbm.at[pl.ds(base0 + (c + 1) * chunk, chunk)],
                    idx_bufs[nxt])
            g_handle.wait()
            if c + 1 < n_chunks:
                if w_handles[nxt] is not None:
                    w_handles[nxt].wait()
                g_handle = pltpu.async_copy(
                    table_hbm.at[idx_bufs[nxt]], row_bufs[nxt], sem_g)
            w_handles[cur] = pltpu.async_copy(
                row_bufs[cur], out_hbm.at[pl.ds(base0 + c * chunk, chunk)],
                sem_w)
        for h in w_handles:
            if h is not None:
                h.wait()

    return gather_k


def kernel(hidden_states, gate_w, gate_up_w, down_w, shared_gate_up_w,
           shared_down_w):
    orig_shape = hidden_states.shape
    x = hidden_states.reshape(-1, D)
    t_tokens = x.shape[0]
    nslot = t_tokens * KTOP
    g_blocks = nslot // BBLK + E
    rows = g_blocks * BBLK

    gw_pad = jnp.zeros((LANES, D), jnp.float32).at[:E].set(gate_w)
    wpad, ipad = pl.pallas_call(
        _gate_body,
        grid=(t_tokens // TBLK,),
        in_specs=[
            pl.BlockSpec((TBLK, D), lambda t: (t, 0)),
            pl.BlockSpec((LANES, D), lambda t: (0, 0)),
        ],
        out_specs=[
            pl.BlockSpec((TBLK, LANES), lambda t: (t, 0)),
            pl.BlockSpec((TBLK, LANES), lambda t: (t, 0)),
        ],
        out_shape=[
            jax.ShapeDtypeStruct((t_tokens, LANES), jnp.float32),
            jax.ShapeDtypeStruct((t_tokens, LANES), jnp.int32),
        ],
    )(x, gw_pad)

    e_flat = ipad[:, :KTOP].reshape(-1)
    one_hot = (e_flat[:, None] == jnp.arange(E)[None, :]).astype(jnp.int32)
    csum = jnp.cumsum(one_hot, axis=0)
    rank_within = jnp.take_along_axis(
        csum, e_flat[:, None], axis=1)[:, 0] - 1
    sizes = csum[-1]
    nb = (sizes + BBLK - 1) // BBLK
    cum_nb = jnp.cumsum(nb)
    block_first = cum_nb - nb
    pos_of_slot = (block_first[e_flat] * BBLK + rank_within
                   ).astype(jnp.int32)
    tok_pad = (jnp.arange(rows, dtype=jnp.int32) % t_tokens).at[
        pos_of_slot].set(
            (jnp.arange(nslot, dtype=jnp.int32) // KTOP))
    block_expert = jnp.clip(
        jnp.searchsorted(cum_nb, jnp.arange(g_blocks), side="right"),
        0, E - 1).astype(jnp.int32)

    xs = _make_sc_gather(rows, 40)(x, tok_pad)

    grid_spec = pltpu.PrefetchScalarGridSpec(
        num_scalar_prefetch=1,
        grid=(g_blocks,),
        in_specs=[
            pl.BlockSpec((BBLK, D), lambda g, be: (g, 0)),
            pl.BlockSpec((1, 2 * FF, D), lambda g, be: (be[g], 0, 0)),
            pl.BlockSpec((1, D, FF), lambda g, be: (be[g], 0, 0)),
        ],
        out_specs=pl.BlockSpec((BBLK, D), lambda g, be: (g, 0)),
    )
    ys = pl.pallas_call(
        _moe_body,
        grid_spec=grid_spec,
        out_shape=jax.ShapeDtypeStruct((rows, D), jnp.float32),
        compiler_params=pltpu.CompilerParams(
            dimension_semantics=("arbitrary",)),
    )(block_expert, xs, gate_up_w, down_w)

    z = _make_sc_gather(nslot, 32)(ys, pos_of_slot)
    z2 = z.reshape(t_tokens, KTOP * D)

    sgw = shared_gate_up_w[:SFF]
    suw = shared_gate_up_w[SFF:]
    ysh = pl.pallas_call(
        _shared_body,
        grid=(t_tokens // TBLK, NCH),
        in_specs=[
            pl.BlockSpec((TBLK, D), lambda t, c: (t, 0)),
            pl.BlockSpec((CH, D), lambda t, c: (c, 0)),
            pl.BlockSpec((CH, D), lambda t, c: (c, 0)),
            pl.BlockSpec((D, CH), lambda t, c: (0, c)),
        ],
        out_specs=pl.BlockSpec((TBLK, D), lambda t, c: (t, 0)),
        out_shape=jax.ShapeDtypeStruct((t_tokens, D), jnp.float32),
        compiler_params=pltpu.CompilerParams(
            dimension_semantics=("parallel", "arbitrary")),
    )(x, sgw, suw, shared_down_w)

    y = pl.pallas_call(
        _combine_body,
        grid=(t_tokens // TBLK,),
        in_specs=[
            pl.BlockSpec((TBLK, D), lambda t: (t, 0)),
            pl.BlockSpec((TBLK, KTOP * D), lambda t: (t, 0)),
            pl.BlockSpec((TBLK, LANES), lambda t: (t, 0)),
        ],
        out_specs=pl.BlockSpec((TBLK, D), lambda t: (t, 0)),
        out_shape=jax.ShapeDtypeStruct((t_tokens, D), jnp.float32),
    )(ysh, z2, wpad)

    return y.reshape(orig_shape)


# speedup vs baseline: 1.1880x; 1.1851x over previous
import functools

import jax
import jax.numpy as jnp
from jax import lax
from jax.experimental import pallas as pl
from jax.experimental.pallas import tpu as pltpu
from jax.experimental.pallas import tpu_sc as plsc

E = 8
KTOP = 2
D = 1024
FF = 1408
SFF = 2816
LANES = 128

NC_SC = 2
NS_SC = 16
NW = NC_SC * NS_SC

BBLK = 256
TBLK = 256
NCH = 2
CH = SFF // NCH


def _gate_body(x_ref, gw_ref, w_ref, i_ref):
    logits = lax.dot_general(
        x_ref[...], gw_ref[...], (((1,), (1,)), ((), ())),
        preferred_element_type=jnp.float32)
    lane = lax.broadcasted_iota(jnp.int32, logits.shape, 1)
    valid = lane < E
    lm = jnp.where(valid, logits, jnp.float32(-1e30))
    m = jnp.max(lm, axis=1, keepdims=True)
    p = jnp.exp(lm - m)
    p = jnp.where(valid, p, 0.0)
    probs = p / jnp.sum(p, axis=1, keepdims=True)
    m1 = jnp.max(probs, axis=1, keepdims=True)
    i1 = jnp.min(jnp.where(probs == m1, lane, LANES), axis=1, keepdims=True)
    probs2 = jnp.where(lane == i1, -1.0, probs)
    m2 = jnp.max(probs2, axis=1, keepdims=True)
    i2 = jnp.min(jnp.where(probs2 == m2, lane, LANES), axis=1, keepdims=True)
    w_ref[...] = jnp.where(lane == 0, m1, jnp.where(lane == 1, m2, 0.0))
    i_ref[...] = jnp.where(lane == 0, i1, jnp.where(lane == 1, i2, 0))


def _moe_body(be_ref, xs_ref, guw_ref, dw_ref, ys_ref):
    del be_ref
    h = lax.dot_general(
        xs_ref[...], guw_ref[0], (((1,), (1,)), ((), ())),
        preferred_element_type=jnp.float32)
    g = h[:, :FF]
    u = h[:, FF:]
    act = (g * lax.logistic(g)) * u
    ys_ref[...] = lax.dot_general(
        act, dw_ref[0], (((1,), (1,)), ((), ())),
        preferred_element_type=jnp.float32)


def _shared_body(x_ref, sg_ref, su_ref, sd_ref, y_ref):
    c = pl.program_id(1)
    x = x_ref[...]
    hg = lax.dot_general(x, sg_ref[...], (((1,), (1,)), ((), ())),
                         preferred_element_type=jnp.float32)
    hu = lax.dot_general(x, su_ref[...], (((1,), (1,)), ((), ())),
                         preferred_element_type=jnp.float32)
    act = (hg * lax.logistic(hg)) * hu
    part = lax.dot_general(act, sd_ref[...], (((1,), (1,)), ((), ())),
                           preferred_element_type=jnp.float32)

    @pl.when(c == 0)
    def _init():
        y_ref[...] = part

    @pl.when(c != 0)
    def _acc():
        y_ref[...] += part


def _combine_body(ysh_ref, z_ref, w_ref, y_ref):
    w = w_ref[...]
    lane = lax.broadcasted_iota(jnp.int32, w.shape, 1)
    w0 = jnp.sum(jnp.where(lane == 0, w, 0.0), axis=1, keepdims=True)
    w1 = jnp.sum(jnp.where(lane == 1, w, 0.0), axis=1, keepdims=True)
    z = z_ref[...]
    y_ref[...] = ysh_ref[...] + w0 * z[:, :D] + w1 * z[:, D:]


def _make_sc_gather(rows_out, chunk):
    per_w = rows_out // NW
    n_chunks = per_w // chunk
    mesh = plsc.VectorSubcoreMesh(
        core_axis_name="c", subcore_axis_name="s",
        num_cores=NC_SC, num_subcores=NS_SC)

    @functools.partial(
        pl.kernel,
        out_type=jax.ShapeDtypeStruct((rows_out, D), jnp.float32),
        mesh=mesh,
        scratch_types=[
            pltpu.VMEM((chunk,), jnp.int32),
            pltpu.VMEM((chunk,), jnp.int32),
            pltpu.VMEM((chunk, D), jnp.float32),
            pltpu.VMEM((chunk, D), jnp.float32),
            pltpu.SemaphoreType.DMA,
            pltpu.SemaphoreType.DMA,
        ],
    )
    def gather_k(table_hbm, idx_hbm, out_hbm, idx_v0, idx_v1, rows_v0,
                 rows_v1, sem_g, sem_w):
        wid = lax.axis_index("s") * NC_SC + lax.axis_index("c")
        idx_bufs = (idx_v0, idx_v1)
        row_bufs = (rows_v0, rows_v1)
        base0 = wid * per_w
        pltpu.sync_copy(idx_hbm.at[pl.ds(base0, chunk)], idx_bufs[0])
        g_handle = pltpu.async_copy(
            table_hbm.at[idx_bufs[0]], row_bufs[0], sem_g)
        w_handles = [None, None]
        for c in range(n_chunks):
            cur = c % 2
            nxt = 1 - cur
            if c + 1 < n_chunks:
                pltpu.sync_copy(
                    idx_hbm.at[pl.ds(base0 + (c + 1) * chunk, chunk)],
                    idx_bufs[nxt])
            g_handle.wait()
            if c + 1 < n_chunks:
                if w_handles[nxt] is not None:
                    w_handles[nxt].wait()
                g_handle = pltpu.async_copy(
                    table_hbm.at[idx_bufs[nxt]], row_bufs[nxt], sem_g)
            w_handles[cur] = pltpu.async_copy(
                row_bufs[cur], out_hbm.at[pl.ds(base0 + c * chunk, chunk)],
                sem_w)
        for h in w_handles:
            if h is not None:
                h.wait()

    return gather_k


def kernel(hidden_states, gate_w, gate_up_w, down_w, shared_gate_up_w,
           shared_down_w):
    orig_shape = hidden_states.shape
    x = hidden_states.reshape(-1, D)
    t_tokens = x.shape[0]
    nslot = t_tokens * KTOP
    g_blocks = nslot // BBLK + E
    rows = g_blocks * BBLK

    gw_pad = jnp.zeros((LANES, D), jnp.float32).at[:E].set(gate_w)
    wpad, ipad = pl.pallas_call(
        _gate_body,
        grid=(t_tokens // TBLK,),
        in_specs=[
            pl.BlockSpec((TBLK, D), lambda t: (t, 0)),
            pl.BlockSpec((LANES, D), lambda t: (0, 0)),
        ],
        out_specs=[
            pl.BlockSpec((TBLK, LANES), lambda t: (t, 0)),
            pl.BlockSpec((TBLK, LANES), lambda t: (t, 0)),
        ],
        out_shape=[
            jax.ShapeDtypeStruct((t_tokens, LANES), jnp.float32),
            jax.ShapeDtypeStruct((t_tokens, LANES), jnp.int32),
        ],
    )(x, gw_pad)

    e_flat = ipad[:, :KTOP].reshape(-1)
    one_hot = (e_flat[:, None] == jnp.arange(E)[None, :]).astype(jnp.int32)
    csum = jnp.cumsum(one_hot, axis=0)
    rank_within = jnp.take_along_axis(
        csum, e_flat[:, None], axis=1)[:, 0] - 1
    sizes = csum[-1]
    nb = (sizes + BBLK - 1) // BBLK
    cum_nb = jnp.cumsum(nb)
    block_first = cum_nb - nb
    pos_of_slot = (block_first[e_flat] * BBLK + rank_within
                   ).astype(jnp.int32)
    tok_pad = (jnp.arange(rows, dtype=jnp.int32) % t_tokens).at[
        pos_of_slot].set(
            (jnp.arange(nslot, dtype=jnp.int32) // KTOP))
    block_expert = jnp.clip(
        jnp.searchsorted(cum_nb, jnp.arange(g_blocks), side="right"),
        0, E - 1).astype(jnp.int32)

    xs = _make_sc_gather(rows, 48)(x, tok_pad)

    grid_spec = pltpu.PrefetchScalarGridSpec(
        num_scalar_prefetch=1,
        grid=(g_blocks,),
        in_specs=[
            pl.BlockSpec((BBLK, D), lambda g, be: (g, 0)),
            pl.BlockSpec((1, 2 * FF, D), lambda g, be: (be[g], 0, 0)),
            pl.BlockSpec((1, D, FF), lambda g, be: (be[g], 0, 0)),
        ],
        out_specs=pl.BlockSpec((BBLK, D), lambda g, be: (g, 0)),
    )
    ys = pl.pallas_call(
        _moe_body,
        grid_spec=grid_spec,
        out_shape=jax.ShapeDtypeStruct((rows, D), jnp.float32),
        compiler_params=pltpu.CompilerParams(
            dimension_semantics=("arbitrary",)),
    )(block_expert, xs, gate_up_w, down_w)

    z = _make_sc_gather(nslot, 32)(ys, pos_of_slot)
    z2 = z.reshape(t_tokens, KTOP * D)

    sgw = shared_gate_up_w[:SFF]
    suw = shared_gate_up_w[SFF:]
    ysh = pl.pallas_call(
        _shared_body,
        grid=(t_tokens // TBLK, NCH),
        in_specs=[
            pl.BlockSpec((TBLK, D), lambda t, c: (t, 0)),
            pl.BlockSpec((CH, D), lambda t, c: (c, 0)),
            pl.BlockSpec((CH, D), lambda t, c: (c, 0)),
            pl.BlockSpec((D, CH), lambda t, c: (0, c)),
        ],
        out_specs=pl.BlockSpec((TBLK, D), lambda t, c: (t, 0)),
        out_shape=jax.ShapeDtypeStruct((t_tokens, D), jnp.float32),
        compiler_params=pltpu.CompilerParams(
            dimension_semantics=("parallel", "arbitrary")),
    )(x, sgw, suw, shared_down_w)

    y = pl.pallas_call(
        _combine_body,
        grid=(t_tokens // TBLK,),
        in_specs=[
            pl.BlockSpec((TBLK, D), lambda t: (t, 0)),
            pl.BlockSpec((TBLK, KTOP * D), lambda t: (t, 0)),
            pl.BlockSpec((TBLK, LANES), lambda t: (t, 0)),
        ],
        out_specs=pl.BlockSpec((TBLK, D), lambda t: (t, 0)),
        out_shape=jax.ShapeDtypeStruct((t_tokens, D), jnp.float32),
    )(ysh, z2, wpad)

    return y.reshape(orig_shape)


# speedup vs baseline: 1.1979x; 1.0083x over previous
import functools

import jax
import jax.numpy as jnp
from jax import lax
from jax.experimental import pallas as pl
from jax.experimental.pallas import tpu as pltpu
from jax.experimental.pallas import tpu_sc as plsc

E = 8
KTOP = 2
D = 1024
FF = 1408
SFF = 2816
LANES = 128

NC_SC = 2
NS_SC = 16
NW = NC_SC * NS_SC

BBLK = 256
TBLK = 256
NCH = 2
CH = SFF // NCH


def _gate_body(x_ref, gw_ref, w_ref, i_ref):
    logits = lax.dot_general(
        x_ref[...], gw_ref[...], (((1,), (1,)), ((), ())),
        preferred_element_type=jnp.float32)
    lane = lax.broadcasted_iota(jnp.int32, logits.shape, 1)
    valid = lane < E
    lm = jnp.where(valid, logits, jnp.float32(-1e30))
    m = jnp.max(lm, axis=1, keepdims=True)
    p = jnp.exp(lm - m)
    p = jnp.where(valid, p, 0.0)
    probs = p / jnp.sum(p, axis=1, keepdims=True)
    m1 = jnp.max(probs, axis=1, keepdims=True)
    i1 = jnp.min(jnp.where(probs == m1, lane, LANES), axis=1, keepdims=True)
    probs2 = jnp.where(lane == i1, -1.0, probs)
    m2 = jnp.max(probs2, axis=1, keepdims=True)
    i2 = jnp.min(jnp.where(probs2 == m2, lane, LANES), axis=1, keepdims=True)
    w_ref[...] = jnp.where(lane == 0, m1, jnp.where(lane == 1, m2, 0.0))
    i_ref[...] = jnp.where(lane == 0, i1, jnp.where(lane == 1, i2, 0))


def _moe_body(be_ref, xs_ref, guw_ref, dw_ref, ys_ref):
    del be_ref
    h = lax.dot_general(
        xs_ref[...], guw_ref[0], (((1,), (1,)), ((), ())),
        preferred_element_type=jnp.float32)
    g = h[:, :FF]
    u = h[:, FF:]
    act = (g * lax.logistic(g)) * u
    ys_ref[...] = lax.dot_general(
        act, dw_ref[0], (((1,), (1,)), ((), ())),
        preferred_element_type=jnp.float32)


def _shared_body(x_ref, sg_ref, su_ref, sd_ref, y_ref):
    c = pl.program_id(1)
    x = x_ref[...]
    hg = lax.dot_general(x, sg_ref[...], (((1,), (1,)), ((), ())),
                         preferred_element_type=jnp.float32)
    hu = lax.dot_general(x, su_ref[...], (((1,), (1,)), ((), ())),
                         preferred_element_type=jnp.float32)
    act = (hg * lax.logistic(hg)) * hu
    part = lax.dot_general(act, sd_ref[...], (((1,), (1,)), ((), ())),
                           preferred_element_type=jnp.float32)

    @pl.when(c == 0)
    def _init():
        y_ref[...] = part

    @pl.when(c != 0)
    def _acc():
        y_ref[...] += part


def _combine_body(ysh_ref, z_ref, w_ref, y_ref):
    w = w_ref[...]
    lane = lax.broadcasted_iota(jnp.int32, w.shape, 1)
    w0 = jnp.sum(jnp.where(lane == 0, w, 0.0), axis=1, keepdims=True)
    w1 = jnp.sum(jnp.where(lane == 1, w, 0.0), axis=1, keepdims=True)
    z = z_ref[...]
    y_ref[...] = ysh_ref[...] + w0 * z[:, :D] + w1 * z[:, D:]


def _make_sc_gather(rows_out, chunk):
    per_w = rows_out // NW
    n_chunks = per_w // chunk
    mesh = plsc.VectorSubcoreMesh(
        core_axis_name="c", subcore_axis_name="s",
        num_cores=NC_SC, num_subcores=NS_SC)

    @functools.partial(
        pl.kernel,
        out_type=jax.ShapeDtypeStruct((rows_out, D), jnp.float32),
        mesh=mesh,
        scratch_types=[
            pltpu.VMEM((chunk,), jnp.int32),
            pltpu.VMEM((chunk,), jnp.int32),
            pltpu.VMEM((chunk, D), jnp.float32),
            pltpu.VMEM((chunk, D), jnp.float32),
            pltpu.SemaphoreType.DMA,
            pltpu.SemaphoreType.DMA,
        ],
    )
    def gather_k(table_hbm, idx_hbm, out_hbm, idx_v0, idx_v1, rows_v0,
                 rows_v1, sem_g, sem_w):
        wid = lax.axis_index("s") * NC_SC + lax.axis_index("c")
        idx_bufs = (idx_v0, idx_v1)
        row_bufs = (rows_v0, rows_v1)
        base0 = wid * per_w
        pltpu.sync_copy(idx_hbm.at[pl.ds(base0, chunk)], idx_bufs[0])
        g_handle = pltpu.async_copy(
            table_hbm.at[idx_bufs[0]], row_bufs[0], sem_g)
        w_handles = [None, None]
        for c in range(n_chunks):
            cur = c % 2
            nxt = 1 - cur
            if c + 1 < n_chunks:
                pltpu.sync_copy(
                    idx_hbm.at[pl.ds(base0 + (c + 1) * chunk, chunk)],
                    idx_bufs[nxt])
            g_handle.wait()
            if c + 1 < n_chunks:
                if w_handles[nxt] is not None:
                    w_handles[nxt].wait()
                g_handle = pltpu.async_copy(
                    table_hbm.at[idx_bufs[nxt]], row_bufs[nxt], sem_g)
            w_handles[cur] = pltpu.async_copy(
                row_bufs[cur], out_hbm.at[pl.ds(base0 + c * chunk, chunk)],
                sem_w)
        for h in w_handles:
            if h is not None:
                h.wait()

    return gather_k


def kernel(hidden_states, gate_w, gate_up_w, down_w, shared_gate_up_w,
           shared_down_w):
    orig_shape = hidden_states.shape
    x = hidden_states.reshape(-1, D)
    t_tokens = x.shape[0]
    nslot = t_tokens * KTOP
    g_blocks = nslot // BBLK + E
    rows = g_blocks * BBLK

    gw_pad = jnp.zeros((LANES, D), jnp.float32).at[:E].set(gate_w)
    wpad, ipad = pl.pallas_call(
        _gate_body,
        grid=(t_tokens // TBLK,),
        in_specs=[
            pl.BlockSpec((TBLK, D), lambda t: (t, 0)),
            pl.BlockSpec((LANES, D), lambda t: (0, 0)),
        ],
        out_specs=[
            pl.BlockSpec((TBLK, LANES), lambda t: (t, 0)),
            pl.BlockSpec((TBLK, LANES), lambda t: (t, 0)),
        ],
        out_shape=[
            jax.ShapeDtypeStruct((t_tokens, LANES), jnp.float32),
            jax.ShapeDtypeStruct((t_tokens, LANES), jnp.int32),
        ],
    )(x, gw_pad)

    e_flat = ipad[:, :KTOP].reshape(-1)
    chunkn = 128
    nchk = nslot // chunkn
    oh3 = (e_flat.reshape(nchk, chunkn, 1) ==
           jnp.arange(E).reshape(1, 1, E)).astype(jnp.float32)
    tri = (jnp.arange(chunkn)[:, None] >= jnp.arange(chunkn)[None, :]
           ).astype(jnp.float32)
    csum_in = jnp.einsum("ij,bjk->bik", tri, oh3,
                         preferred_element_type=jnp.float32)
    totals = csum_in[:, -1, :]
    offs = jnp.cumsum(totals, axis=0) - totals
    csum = (csum_in + offs[:, None, :]).astype(jnp.int32).reshape(nslot, E)
    rank_within = jnp.take_along_axis(
        csum, e_flat[:, None], axis=1)[:, 0] - 1
    sizes = csum[-1]
    nb = (sizes + BBLK - 1) // BBLK
    cum_nb = jnp.cumsum(nb)
    block_first = cum_nb - nb
    pos_of_slot = (block_first[e_flat] * BBLK + rank_within
                   ).astype(jnp.int32)
    tok_pad = (jnp.arange(rows, dtype=jnp.int32) % t_tokens).at[
        pos_of_slot].set(
            (jnp.arange(nslot, dtype=jnp.int32) // KTOP))
    block_expert = jnp.clip(
        jnp.searchsorted(cum_nb, jnp.arange(g_blocks), side="right"),
        0, E - 1).astype(jnp.int32)

    xs = _make_sc_gather(rows, 48)(x, tok_pad)

    grid_spec = pltpu.PrefetchScalarGridSpec(
        num_scalar_prefetch=1,
        grid=(g_blocks,),
        in_specs=[
            pl.BlockSpec((BBLK, D), lambda g, be: (g, 0)),
            pl.BlockSpec((1, 2 * FF, D), lambda g, be: (be[g], 0, 0)),
            pl.BlockSpec((1, D, FF), lambda g, be: (be[g], 0, 0)),
        ],
        out_specs=pl.BlockSpec((BBLK, D), lambda g, be: (g, 0)),
    )
    ys = pl.pallas_call(
        _moe_body,
        grid_spec=grid_spec,
        out_shape=jax.ShapeDtypeStruct((rows, D), jnp.float32),
        compiler_params=pltpu.CompilerParams(
            dimension_semantics=("arbitrary",)),
    )(block_expert, xs, gate_up_w, down_w)

    z = _make_sc_gather(nslot, 32)(ys, pos_of_slot)
    z2 = z.reshape(t_tokens, KTOP * D)

    sgw = shared_gate_up_w[:SFF]
    suw = shared_gate_up_w[SFF:]
    ysh = pl.pallas_call(
        _shared_body,
        grid=(t_tokens // TBLK, NCH),
        in_specs=[
            pl.BlockSpec((TBLK, D), lambda t, c: (t, 0)),
            pl.BlockSpec((CH, D), lambda t, c: (c, 0)),
            pl.BlockSpec((CH, D), lambda t, c: (c, 0)),
            pl.BlockSpec((D, CH), lambda t, c: (0, c)),
        ],
        out_specs=pl.BlockSpec((TBLK, D), lambda t, c: (t, 0)),
        out_shape=jax.ShapeDtypeStruct((t_tokens, D), jnp.float32),
        compiler_params=pltpu.CompilerParams(
            dimension_semantics=("parallel", "arbitrary")),
    )(x, sgw, suw, shared_down_w)

    y = pl.pallas_call(
        _combine_body,
        grid=(t_tokens // TBLK,),
        in_specs=[
            pl.BlockSpec((TBLK, D), lambda t: (t, 0)),
            pl.BlockSpec((TBLK, KTOP * D), lambda t: (t, 0)),
            pl.BlockSpec((TBLK, LANES), lambda t: (t, 0)),
        ],
        out_specs=pl.BlockSpec((TBLK, D), lambda t: (t, 0)),
        out_shape=jax.ShapeDtypeStruct((t_tokens, D), jnp.float32),
    )(ysh, z2, wpad)

    return y.reshape(orig_shape)


# speedup vs baseline: 1.2213x; 1.0195x over previous
import functools

import jax
import jax.numpy as jnp
from jax import lax
from jax.experimental import pallas as pl
from jax.experimental.pallas import tpu as pltpu
from jax.experimental.pallas import tpu_sc as plsc

E = 8
KTOP = 2
D = 1024
FF = 1408
SFF = 2816
LANES = 128

NC_SC = 2
NS_SC = 16
NW = NC_SC * NS_SC

BBLK = 256
TBLK = 256
NCH = 2
CH = SFF // NCH


def _gate_body(x_ref, gw_ref, w_ref, i_ref):
    logits = lax.dot_general(
        x_ref[...], gw_ref[...], (((1,), (1,)), ((), ())),
        preferred_element_type=jnp.float32)
    lane = lax.broadcasted_iota(jnp.int32, logits.shape, 1)
    valid = lane < E
    lm = jnp.where(valid, logits, jnp.float32(-1e30))
    m = jnp.max(lm, axis=1, keepdims=True)
    p = jnp.exp(lm - m)
    p = jnp.where(valid, p, 0.0)
    probs = p / jnp.sum(p, axis=1, keepdims=True)
    m1 = jnp.max(probs, axis=1, keepdims=True)
    i1 = jnp.min(jnp.where(probs == m1, lane, LANES), axis=1, keepdims=True)
    probs2 = jnp.where(lane == i1, -1.0, probs)
    m2 = jnp.max(probs2, axis=1, keepdims=True)
    i2 = jnp.min(jnp.where(probs2 == m2, lane, LANES), axis=1, keepdims=True)
    w_ref[...] = jnp.where(lane == 0, m1, jnp.where(lane == 1, m2, 0.0))
    i_ref[...] = jnp.where(lane == 0, i1, jnp.where(lane == 1, i2, 0))


def _moe_body(be_ref, xs_ref, guw_ref, dw_ref, ys_ref):
    del be_ref
    h = lax.dot_general(
        xs_ref[...], guw_ref[0], (((1,), (1,)), ((), ())),
        preferred_element_type=jnp.float32)
    g = h[:, :FF]
    u = h[:, FF:]
    act = (g * lax.logistic(g)) * u
    ys_ref[...] = lax.dot_general(
        act, dw_ref[0], (((1,), (1,)), ((), ())),
        preferred_element_type=jnp.float32)


def _shared_body(x_ref, sg_ref, su_ref, sd_ref, z_ref, w_ref, y_ref):
    c = pl.program_id(1)
    x = x_ref[...]
    hg = lax.dot_general(x, sg_ref[...], (((1,), (1,)), ((), ())),
                         preferred_element_type=jnp.float32)
    hu = lax.dot_general(x, su_ref[...], (((1,), (1,)), ((), ())),
                         preferred_element_type=jnp.float32)
    act = (hg * lax.logistic(hg)) * hu
    part = lax.dot_general(act, sd_ref[...], (((1,), (1,)), ((), ())),
                           preferred_element_type=jnp.float32)

    @pl.when(c == 0)
    def _init():
        y_ref[...] = part

    @pl.when(c != 0)
    def _acc():
        y_ref[...] += part

    @pl.when(c == NCH - 1)
    def _combine():
        w = w_ref[...]
        lane = lax.broadcasted_iota(jnp.int32, w.shape, 1)
        w0 = jnp.sum(jnp.where(lane == 0, w, 0.0), axis=1, keepdims=True)
        w1 = jnp.sum(jnp.where(lane == 1, w, 0.0), axis=1, keepdims=True)
        z = z_ref[...]
        y_ref[...] += w0 * z[:, :D] + w1 * z[:, D:]


def _make_sc_gather(rows_out, chunk):
    per_w = rows_out // NW
    n_chunks = per_w // chunk
    mesh = plsc.VectorSubcoreMesh(
        core_axis_name="c", subcore_axis_name="s",
        num_cores=NC_SC, num_subcores=NS_SC)

    @functools.partial(
        pl.kernel,
        out_type=jax.ShapeDtypeStruct((rows_out, D), jnp.float32),
        mesh=mesh,
        scratch_types=[
            pltpu.VMEM((chunk,), jnp.int32),
            pltpu.VMEM((chunk,), jnp.int32),
            pltpu.VMEM((chunk, D), jnp.float32),
            pltpu.VMEM((chunk, D), jnp.float32),
            pltpu.SemaphoreType.DMA,
            pltpu.SemaphoreType.DMA,
        ],
    )
    def gather_k(table_hbm, idx_hbm, out_hbm, idx_v0, idx_v1, rows_v0,
                 rows_v1, sem_g, sem_w):
        wid = lax.axis_index("s") * NC_SC + lax.axis_index("c")
        idx_bufs = (idx_v0, idx_v1)
        row_bufs = (rows_v0, rows_v1)
        base0 = wid * per_w
        pltpu.sync_copy(idx_hbm.at[pl.ds(base0, chunk)], idx_bufs[0])
        g_handle = pltpu.async_copy(
            table_hbm.at[idx_bufs[0]], row_bufs[0], sem_g)
        w_handles = [None, None]
        for c in range(n_chunks):
            cur = c % 2
            nxt = 1 - cur
            if c + 1 < n_chunks:
                pltpu.sync_copy(
                    idx_hbm.at[pl.ds(base0 + (c + 1) * chunk, chunk)],
                    idx_bufs[nxt])
            g_handle.wait()
            if c + 1 < n_chunks:
                if w_handles[nxt] is not None:
                    w_handles[nxt].wait()
                g_handle = pltpu.async_copy(
                    table_hbm.at[idx_bufs[nxt]], row_bufs[nxt], sem_g)
            w_handles[cur] = pltpu.async_copy(
                row_bufs[cur], out_hbm.at[pl.ds(base0 + c * chunk, chunk)],
                sem_w)
        for h in w_handles:
            if h is not None:
                h.wait()

    return gather_k


def kernel(hidden_states, gate_w, gate_up_w, down_w, shared_gate_up_w,
           shared_down_w):
    orig_shape = hidden_states.shape
    x = hidden_states.reshape(-1, D)
    t_tokens = x.shape[0]
    nslot = t_tokens * KTOP
    g_blocks = nslot // BBLK + E
    rows = g_blocks * BBLK

    gw_pad = jnp.zeros((LANES, D), jnp.float32).at[:E].set(gate_w)
    wpad, ipad = pl.pallas_call(
        _gate_body,
        grid=(t_tokens // TBLK,),
        in_specs=[
            pl.BlockSpec((TBLK, D), lambda t: (t, 0)),
            pl.BlockSpec((LANES, D), lambda t: (0, 0)),
        ],
        out_specs=[
            pl.BlockSpec((TBLK, LANES), lambda t: (t, 0)),
            pl.BlockSpec((TBLK, LANES), lambda t: (t, 0)),
        ],
        out_shape=[
            jax.ShapeDtypeStruct((t_tokens, LANES), jnp.float32),
            jax.ShapeDtypeStruct((t_tokens, LANES), jnp.int32),
        ],
    )(x, gw_pad)

    e_flat = ipad[:, :KTOP].reshape(-1)
    chunkn = 128
    nchk = nslot // chunkn
    oh3 = (e_flat.reshape(nchk, chunkn, 1) ==
           jnp.arange(E).reshape(1, 1, E)).astype(jnp.float32)
    tri = (jnp.arange(chunkn)[:, None] >= jnp.arange(chunkn)[None, :]
           ).astype(jnp.float32)
    csum_in = jnp.einsum("ij,bjk->bik", tri, oh3,
                         preferred_element_type=jnp.float32)
    totals = csum_in[:, -1, :]
    offs = jnp.cumsum(totals, axis=0) - totals
    csum = (csum_in + offs[:, None, :]).astype(jnp.int32).reshape(nslot, E)
    rank_within = jnp.take_along_axis(
        csum, e_flat[:, None], axis=1)[:, 0] - 1
    sizes = csum[-1]
    nb = (sizes + BBLK - 1) // BBLK
    cum_nb = jnp.cumsum(nb)
    block_first = cum_nb - nb
    pos_of_slot = (block_first[e_flat] * BBLK + rank_within
                   ).astype(jnp.int32)
    tok_pad = (jnp.arange(rows, dtype=jnp.int32) % t_tokens).at[
        pos_of_slot].set(
            (jnp.arange(nslot, dtype=jnp.int32) // KTOP))
    block_expert = jnp.clip(
        jnp.searchsorted(cum_nb, jnp.arange(g_blocks), side="right"),
        0, E - 1).astype(jnp.int32)

    xs = _make_sc_gather(rows, 48)(x, tok_pad)

    grid_spec = pltpu.PrefetchScalarGridSpec(
        num_scalar_prefetch=1,
        grid=(g_blocks,),
        in_specs=[
            pl.BlockSpec((BBLK, D), lambda g, be: (g, 0)),
            pl.BlockSpec((1, 2 * FF, D), lambda g, be: (be[g], 0, 0)),
            pl.BlockSpec((1, D, FF), lambda g, be: (be[g], 0, 0)),
        ],
        out_specs=pl.BlockSpec((BBLK, D), lambda g, be: (g, 0)),
    )
    ys = pl.pallas_call(
        _moe_body,
        grid_spec=grid_spec,
        out_shape=jax.ShapeDtypeStruct((rows, D), jnp.float32),
        compiler_params=pltpu.CompilerParams(
            dimension_semantics=("arbitrary",)),
    )(block_expert, xs, gate_up_w, down_w)

    z = _make_sc_gather(nslot, 32)(ys, pos_of_slot)
    z2 = z.reshape(t_tokens, KTOP * D)

    sgw = shared_gate_up_w[:SFF]
    suw = shared_gate_up_w[SFF:]
    y = pl.pallas_call(
        _shared_body,
        grid=(t_tokens // TBLK, NCH),
        in_specs=[
            pl.BlockSpec((TBLK, D), lambda t, c: (t, 0)),
            pl.BlockSpec((CH, D), lambda t, c: (c, 0)),
            pl.BlockSpec((CH, D), lambda t, c: (c, 0)),
            pl.BlockSpec((D, CH), lambda t, c: (0, c)),
            pl.BlockSpec((TBLK, KTOP * D), lambda t, c: (t, 0)),
            pl.BlockSpec((TBLK, LANES), lambda t, c: (t, 0)),
        ],
        out_specs=pl.BlockSpec((TBLK, D), lambda t, c: (t, 0)),
        out_shape=jax.ShapeDtypeStruct((t_tokens, D), jnp.float32),
        compiler_params=pltpu.CompilerParams(
            dimension_semantics=("parallel", "arbitrary")),
    )(x, sgw, suw, shared_down_w, z2, wpad)

    return y.reshape(orig_shape)
